# baseline (speedup 1.0000x reference)
import math

import numpy as np
import jax
import jax.numpy as jnp
from jax import lax
from jax.experimental import pallas as pl
from jax.experimental.pallas import tpu as pltpu
from jax.experimental.pallas import tpu_sc as plsc

D_MODEL = 768
MAXLEN = 2048
SCALE = float(np.float32(math.sqrt(float(D_MODEL))))

NUM_WORKERS = 32
CHUNK = 64
SPLIT = 1
EPI_ROWS = 2048


def _pos_encoding_np(length: int, depth: int) -> np.ndarray:
    half = depth / 2
    positions = np.arange(length)[:, np.newaxis]
    depths = np.arange(half)[np.newaxis, :] / half
    angle_rates = 1 / 10000 ** (2 * depths)
    angle_rads = positions * angle_rates
    return np.concatenate(
        [np.sin(angle_rads), np.cos(angle_rads)], axis=-1
    ).astype(np.float32)


_POS = _pos_encoding_np(MAXLEN, D_MODEL)

_MESH = plsc.VectorSubcoreMesh(core_axis_name="c", subcore_axis_name="s")


def _sc_gather(table, idx_s, ns):
    per_w = ns // NUM_WORKERS
    n_chunks = per_w // CHUNK

    @pl.kernel(
        out_type=jax.ShapeDtypeStruct((ns, D_MODEL), jnp.float32),
        mesh=_MESH,
        scratch_types=[
            pltpu.VMEM((CHUNK,), jnp.int32),
            pltpu.VMEM((CHUNK,), jnp.int32),
            pltpu.VMEM((2, CHUNK, D_MODEL), jnp.float32),
            pltpu.SemaphoreType.DMA,
            pltpu.SemaphoreType.DMA,
            pltpu.SemaphoreType.DMA,
            pltpu.SemaphoreType.DMA,
            pltpu.SemaphoreType.DMA,
            pltpu.SemaphoreType.DMA,
        ],
    )
    def k(table_hbm, idx_hbm, out_hbm,
          idx0, idx1, rows_v, i0, i1, g0, g1, o0, o1):
        idxc = (idx0, idx1)
        isem = (i0, i1)
        gsem = (g0, g1)
        osem = (o0, o1)
        wid = lax.axis_index("s") * 2 + lax.axis_index("c")
        base = wid * per_w

        def issue_idx(c, b):
            pltpu.async_copy(
                idx_hbm.at[pl.ds(base + c * CHUNK, CHUNK)], idxc[b], isem[b])

        def issue_gather(b):
            pltpu.async_copy(table_hbm.at[idxc[b]], rows_v.at[b], gsem[b])

        issue_idx(0, 0)
        pltpu.make_async_copy(
            idx_hbm.at[pl.ds(base, CHUNK)], idxc[0], isem[0]).wait()
        issue_gather(0)
        if n_chunks > 1:
            issue_idx(1, 1)

        for c in range(n_chunks):
            b = c % 2
            nb = (c + 1) % 2
            if c + 1 < n_chunks:
                pltpu.make_async_copy(
                    idx_hbm.at[pl.ds(base + (c + 1) * CHUNK, CHUNK)],
                    idxc[nb], isem[nb]).wait()
                if c >= 1:
                    pltpu.make_async_copy(
                        rows_v.at[nb],
                        out_hbm.at[pl.ds(base + (c - 1) * CHUNK, CHUNK)],
                        osem[nb]).wait()
                issue_gather(nb)

            pltpu.make_async_copy(
                table_hbm.at[idxc[b]], rows_v.at[b], gsem[b]).wait()
            if c + 2 < n_chunks:
                issue_idx(c + 2, b)

            pltpu.async_copy(
                rows_v.at[b],
                out_hbm.at[pl.ds(base + c * CHUNK, CHUNK)],
                osem[b])

        for c in (n_chunks - 2, n_chunks - 1):
            if c >= 0:
                pltpu.make_async_copy(
                    rows_v.at[c % 2],
                    out_hbm.at[pl.ds(base + c * CHUNK, CHUNK)],
                    osem[c % 2]).wait()

    return k(table, idx_s)


def _tc_epilogue(g_s, pos, dest, s, ns, n):
    pos_blocks = MAXLEN // EPI_ROWS
    reps = ns // MAXLEN
    row0 = s * ns // EPI_ROWS

    def body(*refs):
        g_ref, p_ref, o_ref = refs[-3], refs[-2], refs[-1]
        o_ref[...] = g_ref[...] * SCALE + p_ref[...]

    in_specs = [
        pl.BlockSpec((EPI_ROWS, D_MODEL),
                     lambda j, r: (r * pos_blocks + j, 0)),
        pl.BlockSpec((EPI_ROWS, D_MODEL), lambda j, r: (j, 0)),
    ]
    operands = [g_s, pos]
    io_aliases = {}
    if dest is not None:
        in_specs = [pl.BlockSpec(memory_space=pl.ANY)] + in_specs
        operands = [dest] + operands
        io_aliases = {0: 0}

    return pl.pallas_call(
        body,
        grid=(pos_blocks, reps),
        in_specs=in_specs,
        out_specs=pl.BlockSpec((EPI_ROWS, D_MODEL),
                               lambda j, r: (row0 + r * pos_blocks + j, 0)),
        out_shape=jax.ShapeDtypeStruct((n, D_MODEL), jnp.float32),
        input_output_aliases=io_aliases,
    )(*operands)


def kernel(x, table):
    batch, length = x.shape
    n = batch * length
    ns = n // SPLIT
    idx = x.reshape(n).astype(jnp.int32)
    pos = jnp.asarray(_POS[:length])

    @jax.jit
    def run(table, idx, pos):
        gathered = [
            _sc_gather(table, idx[s * ns:(s + 1) * ns], ns)
            for s in range(SPLIT)
        ]
        out = None
        for s in range(SPLIT):
            out = _tc_epilogue(gathered[s], pos, out, s, ns, n)
        return out

    return run(table, idx, pos).reshape(batch, length, D_MODEL)


# speedup vs baseline: 1.0152x; 1.0152x over previous
import math

import numpy as np
import jax
import jax.numpy as jnp
from jax import lax
from jax.experimental import pallas as pl
from jax.experimental.pallas import tpu as pltpu
from jax.experimental.pallas import tpu_sc as plsc

D_MODEL = 768
MAXLEN = 2048
SCALE = float(np.float32(math.sqrt(float(D_MODEL))))

NUM_WORKERS = 32
CHUNK = 32
NBUF = 4
AHEAD = 2
SPLIT = 1
EPI_ROWS = 2048


def _pos_encoding_np(length: int, depth: int) -> np.ndarray:
    half = depth / 2
    positions = np.arange(length)[:, np.newaxis]
    depths = np.arange(half)[np.newaxis, :] / half
    angle_rates = 1 / 10000 ** (2 * depths)
    angle_rads = positions * angle_rates
    return np.concatenate(
        [np.sin(angle_rads), np.cos(angle_rads)], axis=-1
    ).astype(np.float32)


_POS = _pos_encoding_np(MAXLEN, D_MODEL)

_MESH = plsc.VectorSubcoreMesh(core_axis_name="c", subcore_axis_name="s")


def _sc_gather(table, idx_s, ns):
    per_w = ns // NUM_WORKERS
    n_chunks = per_w // CHUNK

    @pl.kernel(
        out_type=jax.ShapeDtypeStruct((ns, D_MODEL), jnp.float32),
        mesh=_MESH,
        scratch_types=[
            pltpu.VMEM((NBUF, CHUNK), jnp.int32),
            pltpu.VMEM((NBUF, CHUNK, D_MODEL), jnp.float32),
        ]
        + [pltpu.SemaphoreType.DMA] * (3 * NBUF),
    )
    def k(table_hbm, idx_hbm, out_hbm, idxc, rows_v, *sems):
        isem = sems[:NBUF]
        gsem = sems[NBUF:2 * NBUF]
        osem = sems[2 * NBUF:]
        wid = lax.axis_index("s") * 2 + lax.axis_index("c")
        base = wid * per_w

        def issue_idx(c):
            b = c % NBUF
            pltpu.async_copy(
                idx_hbm.at[pl.ds(base + c * CHUNK, CHUNK)],
                idxc.at[b], isem[b])

        def wait_idx(c):
            b = c % NBUF
            pltpu.make_async_copy(
                idx_hbm.at[pl.ds(base + c * CHUNK, CHUNK)],
                idxc.at[b], isem[b]).wait()

        def issue_gather(c):
            b = c % NBUF
            pltpu.async_copy(
                table_hbm.at[idxc.at[b]], rows_v.at[b], gsem[b])

        def wait_gather(c):
            b = c % NBUF
            pltpu.make_async_copy(
                table_hbm.at[idxc.at[b]], rows_v.at[b], gsem[b]).wait()

        def issue_out(c):
            b = c % NBUF
            pltpu.async_copy(
                rows_v.at[b],
                out_hbm.at[pl.ds(base + c * CHUNK, CHUNK)], osem[b])

        def wait_out(c):
            b = c % NBUF
            pltpu.make_async_copy(
                rows_v.at[b],
                out_hbm.at[pl.ds(base + c * CHUNK, CHUNK)], osem[b]).wait()

        for c in range(min(NBUF, n_chunks)):
            issue_idx(c)
        for c in range(min(AHEAD, n_chunks)):
            wait_idx(c)
            issue_gather(c)

        for c in range(n_chunks):
            nxt = c + AHEAD
            if nxt < n_chunks:
                wait_idx(nxt)
                if nxt >= NBUF:
                    wait_out(nxt - NBUF)
                issue_gather(nxt)

            wait_gather(c)
            if c + NBUF < n_chunks:
                issue_idx(c + NBUF)
            issue_out(c)

        for c in range(max(0, n_chunks - NBUF), n_chunks):
            wait_out(c)

    return k(table, idx_s)


def _tc_epilogue(g_s, pos, dest, s, ns, n):
    pos_blocks = MAXLEN // EPI_ROWS
    reps = ns // MAXLEN
    row0 = s * ns // EPI_ROWS

    def body(*refs):
        g_ref, p_ref, o_ref = refs[-3], refs[-2], refs[-1]
        o_ref[...] = g_ref[...] * SCALE + p_ref[...]

    in_specs = [
        pl.BlockSpec((EPI_ROWS, D_MODEL),
                     lambda j, r: (r * pos_blocks + j, 0)),
        pl.BlockSpec((EPI_ROWS, D_MODEL), lambda j, r: (j, 0)),
    ]
    operands = [g_s, pos]
    io_aliases = {}
    if dest is not None:
        in_specs = [pl.BlockSpec(memory_space=pl.ANY)] + in_specs
        operands = [dest] + operands
        io_aliases = {0: 0}

    return pl.pallas_call(
        body,
        grid=(pos_blocks, reps),
        in_specs=in_specs,
        out_specs=pl.BlockSpec((EPI_ROWS, D_MODEL),
                               lambda j, r: (row0 + r * pos_blocks + j, 0)),
        out_shape=jax.ShapeDtypeStruct((n, D_MODEL), jnp.float32),
        input_output_aliases=io_aliases,
    )(*operands)


def kernel(x, table):
    batch, length = x.shape
    n = batch * length
    ns = n // SPLIT
    idx = x.reshape(n).astype(jnp.int32)
    pos = jnp.asarray(_POS[:length])

    @jax.jit
    def run(table, idx, pos):
        gathered = [
            _sc_gather(table, idx[s * ns:(s + 1) * ns], ns)
            for s in range(SPLIT)
        ]
        out = None
        for s in range(SPLIT):
            out = _tc_epilogue(gathered[s], pos, out, s, ns, n)
        return out

    return run(table, idx, pos).reshape(batch, length, D_MODEL)
